# baseline (device time: 71782 ns/iter reference)
import jax
import jax.numpy as jnp
from jax import lax
from jax.experimental import pallas as pl
from jax.experimental.pallas import tpu as pltpu

N_DEV = 8
B = 2
SQ = 256
D = 768
HQ_LOC = 8
HKV_LOC = 2
GROUP = 4
DH = 64
SKV = 512
ROWS = B * SQ
CHUNK = ROWS // N_DEV
N_STEPS = N_DEV - 1
SCALE = 0.125


def kernel(x, Wq, Wo, K_ext, V_ext):
    my_i = lax.axis_index("i")
    k_loc = lax.dynamic_slice_in_dim(K_ext, my_i * HKV_LOC, HKV_LOC, axis=2)
    v_loc = lax.dynamic_slice_in_dim(V_ext, my_i * HKV_LOC, HKV_LOC, axis=2)
    k_loc = jnp.transpose(k_loc, (0, 2, 1, 3))
    v_loc = jnp.transpose(v_loc, (0, 2, 1, 3))
    x2 = x.reshape(ROWS, D)

    def body(x_ref, wq_ref, wo_ref, k_ref, v_ref, out_ref,
             acc_ref, stage_ref, comm_ref, send_sems, recv_sems):
        i = lax.axis_index("i")
        right = lax.rem(i + 1, N_DEV)
        left = lax.rem(i + N_DEV - 1, N_DEV)

        barrier = pltpu.get_barrier_semaphore()
        for nbr in (left, right):
            pl.semaphore_signal(barrier, inc=1, device_id=(nbr,),
                                device_id_type=pl.DeviceIdType.MESH)
        pl.semaphore_wait(barrier, 2)

        xb = x_ref[...].astype(jnp.bfloat16)
        wq = wq_ref[...].astype(jnp.bfloat16)
        q = lax.dot_general(xb, wq, (((1,), (0,)), ((), ())),
                            preferred_element_type=jnp.float32)
        q = q.astype(jnp.bfloat16)

        cols = []
        for h in range(HQ_LOC):
            c = h // GROUP
            outs_b = []
            for b in range(B):
                qh = q[b * SQ:(b + 1) * SQ, h * DH:(h + 1) * DH]
                kc = k_ref[b, c].astype(jnp.bfloat16)
                vc = v_ref[b, c].astype(jnp.bfloat16)
                s = lax.dot_general(qh, kc, (((1,), (1,)), ((), ())),
                                    preferred_element_type=jnp.float32)
                s = s * SCALE
                m = jnp.max(s, axis=-1, keepdims=True)
                p = jnp.exp(s - m)
                l = jnp.sum(p, axis=-1, keepdims=True)
                o = lax.dot_general(p.astype(jnp.bfloat16), vc,
                                    (((1,), (0,)), ((), ())),
                                    preferred_element_type=jnp.float32)
                outs_b.append(o / l)
            cols.append(jnp.concatenate(outs_b, axis=0))
        attn = jnp.concatenate(cols, axis=1).astype(jnp.bfloat16)
        wo = wo_ref[...].astype(jnp.bfloat16)
        acc_ref[...] = lax.dot_general(attn, wo, (((1,), (0,)), ((), ())),
                                       preferred_element_type=jnp.float32)

        for s_ in range(N_STEPS):
            idx_send = lax.rem(i - s_ + N_DEV, N_DEV)
            if s_ == 0:
                stage_ref[...] = acc_ref[pl.ds(idx_send * CHUNK, CHUNK), :]
            rdma = pltpu.make_async_remote_copy(
                src_ref=stage_ref,
                dst_ref=comm_ref.at[s_],
                send_sem=send_sems.at[s_],
                recv_sem=recv_sems.at[s_],
                device_id=(right,),
                device_id_type=pl.DeviceIdType.MESH,
            )
            rdma.start()
            rdma.wait()
            idx_recv = lax.rem(i - s_ - 1 + N_DEV, N_DEV)
            summed = comm_ref[s_] + acc_ref[pl.ds(idx_recv * CHUNK, CHUNK), :]
            if s_ < N_STEPS - 1:
                stage_ref[...] = summed
            else:
                acc_ref[pl.ds(idx_recv * CHUNK, CHUNK), :] = summed

        for t in range(N_STEPS):
            idx_send = lax.rem(i + 1 - t + N_DEV, N_DEV)
            stage_ref[...] = acc_ref[pl.ds(idx_send * CHUNK, CHUNK), :]
            rdma = pltpu.make_async_remote_copy(
                src_ref=stage_ref,
                dst_ref=comm_ref.at[N_STEPS + t],
                send_sem=send_sems.at[N_STEPS + t],
                recv_sem=recv_sems.at[N_STEPS + t],
                device_id=(right,),
                device_id_type=pl.DeviceIdType.MESH,
            )
            rdma.start()
            rdma.wait()
            idx_recv = lax.rem(i - t + N_DEV, N_DEV)
            acc_ref[pl.ds(idx_recv * CHUNK, CHUNK), :] = comm_ref[N_STEPS + t]

        out_ref[...] = acc_ref[...].reshape(B, SQ, D)

    return pl.pallas_call(
        body,
        out_shape=jax.ShapeDtypeStruct((B, SQ, D), jnp.float32),
        in_specs=[pl.BlockSpec(memory_space=pltpu.VMEM)] * 5,
        out_specs=pl.BlockSpec(memory_space=pltpu.VMEM),
        scratch_shapes=[
            pltpu.VMEM((ROWS, D), jnp.float32),
            pltpu.VMEM((CHUNK, D), jnp.float32),
            pltpu.VMEM((2 * N_STEPS, CHUNK, D), jnp.float32),
            pltpu.SemaphoreType.DMA((2 * N_STEPS,)),
            pltpu.SemaphoreType.DMA((2 * N_STEPS,)),
        ],
        compiler_params=pltpu.CompilerParams(collective_id=0),
    )(x2, Wq, Wo, k_loc, v_loc)


# device time: 31555 ns/iter; 2.2748x vs baseline; 2.2748x over previous
import jax
import jax.numpy as jnp
from jax import lax
from jax.experimental import pallas as pl
from jax.experimental.pallas import tpu as pltpu

N_DEV = 8
B = 2
SQ = 256
D = 768
HQ_LOC = 8
HKV_LOC = 2
GROUP = 4
DH = 64
SKV = 512
ROWS = B * SQ
CHUNK = ROWS // N_DEV
N_PEERS = N_DEV - 1
SCALE = 0.125


def kernel(x, Wq, Wo, K_ext, V_ext):
    my_i = lax.axis_index("i")
    k_loc = lax.dynamic_slice_in_dim(K_ext, my_i * HKV_LOC, HKV_LOC, axis=2)
    v_loc = lax.dynamic_slice_in_dim(V_ext, my_i * HKV_LOC, HKV_LOC, axis=2)
    k_loc = jnp.transpose(k_loc, (0, 2, 1, 3))
    v_loc = jnp.transpose(v_loc, (0, 2, 1, 3))
    x2 = x.reshape(ROWS, D)

    def body(x_ref, wq_ref, wo_ref, k_ref, v_ref, out_ref,
             acc_ref, stage_ref, red_ref, rs_ref, ag_ref,
             rs_send_sems, rs_recv_sems, ag_send_sems, ag_recv_sems):
        i = lax.axis_index("i")

        barrier = pltpu.get_barrier_semaphore()
        for o in range(1, N_DEV):
            pl.semaphore_signal(barrier, inc=1,
                                device_id=(lax.rem(i + o, N_DEV),),
                                device_id_type=pl.DeviceIdType.MESH)
        pl.semaphore_wait(barrier, N_PEERS)

        xb = x_ref[...].astype(jnp.bfloat16)
        wq = wq_ref[...].astype(jnp.bfloat16)
        q = lax.dot_general(xb, wq, (((1,), (0,)), ((), ())),
                            preferred_element_type=jnp.float32)
        q = q.astype(jnp.bfloat16)

        cols = []
        for h in range(HQ_LOC):
            c = h // GROUP
            outs_b = []
            for b in range(B):
                qh = q[b * SQ:(b + 1) * SQ, h * DH:(h + 1) * DH]
                kc = k_ref[b, c].astype(jnp.bfloat16)
                vc = v_ref[b, c].astype(jnp.bfloat16)
                s = lax.dot_general(qh, kc, (((1,), (1,)), ((), ())),
                                    preferred_element_type=jnp.float32)
                s = s * SCALE
                m = jnp.max(s, axis=-1, keepdims=True)
                p = jnp.exp(s - m)
                l = jnp.sum(p, axis=-1, keepdims=True)
                o = lax.dot_general(p.astype(jnp.bfloat16), vc,
                                    (((1,), (0,)), ((), ())),
                                    preferred_element_type=jnp.float32)
                outs_b.append(o / l)
            cols.append(jnp.concatenate(outs_b, axis=0))
        attn = jnp.concatenate(cols, axis=1).astype(jnp.bfloat16)
        wo = wo_ref[...].astype(jnp.bfloat16)
        acc_ref[...] = lax.dot_general(attn, wo, (((1,), (0,)), ((), ())),
                                       preferred_element_type=jnp.float32)

        rs_rdmas = []
        for o in range(1, N_DEV):
            p = lax.rem(i + o, N_DEV)
            stage_ref[o - 1] = (
                acc_ref[pl.ds(p * CHUNK, CHUNK), :].astype(jnp.bfloat16))
            rdma = pltpu.make_async_remote_copy(
                src_ref=stage_ref.at[o - 1],
                dst_ref=rs_ref.at[o - 1],
                send_sem=rs_send_sems.at[o - 1],
                recv_sem=rs_recv_sems.at[o - 1],
                device_id=(p,),
                device_id_type=pl.DeviceIdType.MESH,
            )
            rdma.start()
            rs_rdmas.append(rdma)

        red = acc_ref[pl.ds(i * CHUNK, CHUNK), :]
        for o in range(1, N_DEV):
            rs_rdmas[o - 1].wait_recv()
            red = red + rs_ref[o - 1].astype(jnp.float32)
        acc_ref[pl.ds(i * CHUNK, CHUNK), :] = red
        red_ref[...] = red.astype(jnp.bfloat16)

        ag_rdmas = []
        for o in range(1, N_DEV):
            rdma = pltpu.make_async_remote_copy(
                src_ref=red_ref,
                dst_ref=ag_ref.at[o - 1],
                send_sem=ag_send_sems.at[o - 1],
                recv_sem=ag_recv_sems.at[o - 1],
                device_id=(lax.rem(i + o, N_DEV),),
                device_id_type=pl.DeviceIdType.MESH,
            )
            rdma.start()
            ag_rdmas.append(rdma)

        for o in range(1, N_DEV):
            ag_rdmas[o - 1].wait_recv()
            idx = lax.rem(i - o + N_DEV, N_DEV)
            acc_ref[pl.ds(idx * CHUNK, CHUNK), :] = (
                ag_ref[o - 1].astype(jnp.float32))

        out_ref[...] = acc_ref[...].reshape(B, SQ, D)

        for o in range(1, N_DEV):
            rs_rdmas[o - 1].wait_send()
            ag_rdmas[o - 1].wait_send()

    return pl.pallas_call(
        body,
        out_shape=jax.ShapeDtypeStruct((B, SQ, D), jnp.float32),
        in_specs=[pl.BlockSpec(memory_space=pltpu.VMEM)] * 5,
        out_specs=pl.BlockSpec(memory_space=pltpu.VMEM),
        scratch_shapes=[
            pltpu.VMEM((ROWS, D), jnp.float32),
            pltpu.VMEM((N_PEERS, CHUNK, D), jnp.bfloat16),
            pltpu.VMEM((CHUNK, D), jnp.bfloat16),
            pltpu.VMEM((N_PEERS, CHUNK, D), jnp.bfloat16),
            pltpu.VMEM((N_PEERS, CHUNK, D), jnp.bfloat16),
            pltpu.SemaphoreType.DMA((N_PEERS,)),
            pltpu.SemaphoreType.DMA((N_PEERS,)),
            pltpu.SemaphoreType.DMA((N_PEERS,)),
            pltpu.SemaphoreType.DMA((N_PEERS,)),
        ],
        compiler_params=pltpu.CompilerParams(collective_id=0),
    )(x2, Wq, Wo, k_loc, v_loc)


# device time: 27585 ns/iter; 2.6022x vs baseline; 1.1439x over previous
import jax
import jax.numpy as jnp
from jax import lax
from jax.experimental import pallas as pl
from jax.experimental.pallas import tpu as pltpu

N_DEV = 8
B = 2
SQ = 256
D = 768
HQ_LOC = 8
HKV_LOC = 2
GROUP = 4
DH = 64
SKV = 512
ROWS = B * SQ
CHUNK = ROWS // N_DEV
CPB = SQ // CHUNK
N_PEERS = N_DEV - 1
SCALE = 0.125


def kernel(x, Wq, Wo, K_ext, V_ext):
    my_i = lax.axis_index("i")
    k_loc = lax.dynamic_slice_in_dim(K_ext, my_i * HKV_LOC, HKV_LOC, axis=2)
    v_loc = lax.dynamic_slice_in_dim(V_ext, my_i * HKV_LOC, HKV_LOC, axis=2)
    k_loc = jnp.transpose(k_loc, (0, 2, 1, 3))
    v_loc = jnp.transpose(v_loc, (0, 2, 1, 3))
    x2 = x.reshape(ROWS, D)

    def body(x_ref, wq_ref, wo_ref, k_ref, v_ref, out_ref,
             stage_ref, red_ref, rs_ref, ag_ref,
             rs_send, rs_recv, ag_send, ag_recv):
        i = lax.axis_index("i")

        barrier = pltpu.get_barrier_semaphore()
        for o in range(1, N_DEV):
            pl.semaphore_signal(barrier, inc=1,
                                device_id=(lax.rem(i + o, N_DEV),),
                                device_id_type=pl.DeviceIdType.MESH)

        xb = x_ref[...].astype(jnp.bfloat16)
        wq = wq_ref[...].astype(jnp.bfloat16)
        q = lax.dot_general(xb, wq, (((1,), (0,)), ((), ())),
                            preferred_element_type=jnp.float32)
        q = q.astype(jnp.bfloat16)
        wo = wo_ref[...].astype(jnp.bfloat16)

        def partial_batch(b):
            cols = []
            for h in range(HQ_LOC):
                c = h // GROUP
                qh = q[b * SQ:(b + 1) * SQ, h * DH:(h + 1) * DH]
                kc = k_ref[b, c].astype(jnp.bfloat16)
                vc = v_ref[b, c].astype(jnp.bfloat16)
                s = lax.dot_general(qh, kc, (((1,), (1,)), ((), ())),
                                    preferred_element_type=jnp.float32)
                s = s * SCALE
                m = jnp.max(s, axis=-1, keepdims=True)
                p = jnp.exp(s - m)
                l = jnp.sum(p, axis=-1, keepdims=True)
                o_ = lax.dot_general(p.astype(jnp.bfloat16), vc,
                                     (((1,), (0,)), ((), ())),
                                     preferred_element_type=jnp.float32)
                cols.append(o_ / l)
            attn_b = jnp.concatenate(cols, axis=1).astype(jnp.bfloat16)
            return lax.dot_general(attn_b, wo, (((1,), (0,)), ((), ())),
                                   preferred_element_type=jnp.float32)

        def rs_send_chunk(c):
            @pl.when(i != c)
            def _():
                rdma = pltpu.make_async_remote_copy(
                    src_ref=stage_ref.at[c],
                    dst_ref=rs_ref.at[i],
                    send_sem=rs_send.at[c],
                    recv_sem=rs_recv.at[i],
                    device_id=(c,),
                    device_id_type=pl.DeviceIdType.MESH,
                )
                rdma.start()

        part0 = partial_batch(0)
        for c in range(CPB):
            stage_ref[c] = part0[c * CHUNK:(c + 1) * CHUNK, :].astype(
                jnp.bfloat16)
        pl.semaphore_wait(barrier, N_PEERS)
        for c in range(CPB):
            rs_send_chunk(c)

        part1 = partial_batch(1)
        for c in range(CPB, N_DEV):
            stage_ref[c] = part1[(c - CPB) * CHUNK:(c - CPB + 1) * CHUNK,
                                 :].astype(jnp.bfloat16)
        for c in range(CPB, N_DEV):
            rs_send_chunk(c)

        rs_ref[pl.ds(i, 1)] = stage_ref[pl.ds(i, 1)]

        for j in range(N_DEV):
            @pl.when(i != j)
            def _():
                rdma = pltpu.make_async_remote_copy(
                    src_ref=stage_ref.at[j], dst_ref=rs_ref.at[j],
                    send_sem=rs_send.at[j], recv_sem=rs_recv.at[j],
                    device_id=(j,), device_id_type=pl.DeviceIdType.MESH,
                )
                rdma.wait_recv()
        red = rs_ref[0].astype(jnp.float32)
        for j in range(1, N_DEV):
            red = red + rs_ref[j].astype(jnp.float32)
        red_ref[...] = red.astype(jnp.bfloat16)

        for j in range(N_DEV):
            @pl.when(i != j)
            def _():
                rdma = pltpu.make_async_remote_copy(
                    src_ref=red_ref,
                    dst_ref=ag_ref.at[i],
                    send_sem=ag_send.at[j],
                    recv_sem=ag_recv.at[i],
                    device_id=(j,),
                    device_id_type=pl.DeviceIdType.MESH,
                )
                rdma.start()
        out_ref[pl.ds(i * CHUNK, CHUNK), :] = red

        for j in range(N_DEV):
            @pl.when(i != j)
            def _():
                rdma = pltpu.make_async_remote_copy(
                    src_ref=red_ref, dst_ref=ag_ref.at[j],
                    send_sem=ag_send.at[j], recv_sem=ag_recv.at[j],
                    device_id=(j,), device_id_type=pl.DeviceIdType.MESH,
                )
                rdma.wait_recv()
                out_ref[j * CHUNK:(j + 1) * CHUNK, :] = (
                    ag_ref[j].astype(jnp.float32))

        for j in range(N_DEV):
            @pl.when(i != j)
            def _():
                r1 = pltpu.make_async_remote_copy(
                    src_ref=stage_ref.at[j], dst_ref=rs_ref.at[j],
                    send_sem=rs_send.at[j], recv_sem=rs_recv.at[j],
                    device_id=(j,), device_id_type=pl.DeviceIdType.MESH,
                )
                r1.wait_send()
                r2 = pltpu.make_async_remote_copy(
                    src_ref=red_ref, dst_ref=ag_ref.at[j],
                    send_sem=ag_send.at[j], recv_sem=ag_recv.at[j],
                    device_id=(j,), device_id_type=pl.DeviceIdType.MESH,
                )
                r2.wait_send()

    out2 = pl.pallas_call(
        body,
        out_shape=jax.ShapeDtypeStruct((ROWS, D), jnp.float32),
        in_specs=[pl.BlockSpec(memory_space=pltpu.VMEM)] * 5,
        out_specs=pl.BlockSpec(memory_space=pltpu.VMEM),
        scratch_shapes=[
            pltpu.VMEM((N_DEV, CHUNK, D), jnp.bfloat16),
            pltpu.VMEM((CHUNK, D), jnp.bfloat16),
            pltpu.VMEM((N_DEV, CHUNK, D), jnp.bfloat16),
            pltpu.VMEM((N_DEV, CHUNK, D), jnp.bfloat16),
            pltpu.SemaphoreType.DMA((N_DEV,)),
            pltpu.SemaphoreType.DMA((N_DEV,)),
            pltpu.SemaphoreType.DMA((N_DEV,)),
            pltpu.SemaphoreType.DMA((N_DEV,)),
        ],
        compiler_params=pltpu.CompilerParams(collective_id=0),
    )(x2, Wq, Wo, k_loc, v_loc)
    return out2.reshape(B, SQ, D)


# device time: 25287 ns/iter; 2.8387x vs baseline; 1.0909x over previous
import jax
import jax.numpy as jnp
from jax import lax
from jax.experimental import pallas as pl
from jax.experimental.pallas import tpu as pltpu

N_DEV = 8
B = 2
SQ = 256
D = 768
HQ_LOC = 8
HKV_LOC = 2
GROUP = 4
DH = 64
SKV = 512
HC = SQ // N_DEV
N_PEERS = N_DEV - 1
SCALE = 0.125


def kernel(x, Wq, Wo, K_ext, V_ext):
    my_i = lax.axis_index("i")
    k_loc = lax.dynamic_slice_in_dim(K_ext, my_i * HKV_LOC, HKV_LOC, axis=2)
    v_loc = lax.dynamic_slice_in_dim(V_ext, my_i * HKV_LOC, HKV_LOC, axis=2)
    k_loc = jnp.transpose(k_loc, (0, 2, 1, 3))
    v_loc = jnp.transpose(v_loc, (0, 2, 1, 3))
    x2 = x.reshape(B * SQ, D)

    def body(x_ref, wq_ref, wo_ref, k_ref, v_ref, out_ref,
             stage_ref, red_ref, rs_ref, ag_ref,
             rs_send, rs_recv, ag_send, ag_recv):
        i = lax.axis_index("i")

        barrier = pltpu.get_barrier_semaphore()
        for o in range(1, N_DEV):
            pl.semaphore_signal(barrier, inc=1,
                                device_id=(lax.rem(i + o, N_DEV),),
                                device_id_type=pl.DeviceIdType.MESH)

        xb = x_ref[...].astype(jnp.bfloat16)
        wq = wq_ref[...].astype(jnp.bfloat16)
        q = lax.dot_general(xb, wq, (((1,), (0,)), ((), ())),
                            preferred_element_type=jnp.float32)
        q = q.astype(jnp.bfloat16)
        wo = wo_ref[...].astype(jnp.bfloat16)

        def partial_batch(b):
            cols = []
            for h in range(HQ_LOC):
                c = h // GROUP
                qh = q[b * SQ:(b + 1) * SQ, h * DH:(h + 1) * DH]
                kc = k_ref[b, c].astype(jnp.bfloat16)
                vc = v_ref[b, c].astype(jnp.bfloat16)
                s = lax.dot_general(qh, kc, (((1,), (1,)), ((), ())),
                                    preferred_element_type=jnp.float32)
                s = s * SCALE
                m = jnp.max(s, axis=-1, keepdims=True)
                p = jnp.exp(s - m)
                l = jnp.sum(p, axis=-1, keepdims=True)
                o_ = lax.dot_general(p.astype(jnp.bfloat16), vc,
                                     (((1,), (0,)), ((), ())),
                                     preferred_element_type=jnp.float32)
                cols.append(o_ / l)
            attn_b = jnp.concatenate(cols, axis=1).astype(jnp.bfloat16)
            return lax.dot_general(attn_b, wo, (((1,), (0,)), ((), ())),
                                   preferred_element_type=jnp.float32)

        for b in range(B):
            part = partial_batch(b)
            for c in range(N_DEV):
                stage_ref[b * N_DEV + c] = part[c * HC:(c + 1) * HC, :].astype(
                    jnp.bfloat16)
            if b == 0:
                pl.semaphore_wait(barrier, N_PEERS)
            for c in range(N_DEV):
                @pl.when(i != c)
                def _():
                    rdma = pltpu.make_async_remote_copy(
                        src_ref=stage_ref.at[b * N_DEV + c],
                        dst_ref=rs_ref.at[b * N_DEV + i],
                        send_sem=rs_send.at[b * N_DEV + c],
                        recv_sem=rs_recv.at[b * N_DEV + i],
                        device_id=(c,),
                        device_id_type=pl.DeviceIdType.MESH,
                    )
                    rdma.start()

        for b in range(B):
            rs_ref[pl.ds(b * N_DEV + i, 1)] = stage_ref[pl.ds(b * N_DEV + i, 1)]
            for j in range(N_DEV):
                @pl.when(i != j)
                def _():
                    rdma = pltpu.make_async_remote_copy(
                        src_ref=stage_ref.at[b * N_DEV + j],
                        dst_ref=rs_ref.at[b * N_DEV + j],
                        send_sem=rs_send.at[b * N_DEV + j],
                        recv_sem=rs_recv.at[b * N_DEV + j],
                        device_id=(j,), device_id_type=pl.DeviceIdType.MESH,
                    )
                    rdma.wait_recv()
            red = rs_ref[b * N_DEV].astype(jnp.float32)
            for j in range(1, N_DEV):
                red = red + rs_ref[b * N_DEV + j].astype(jnp.float32)
            red_ref[b] = red.astype(jnp.bfloat16)
            out_ref[pl.ds(b * SQ + i * HC, HC), :] = red
            for j in range(N_DEV):
                @pl.when(i != j)
                def _():
                    rdma = pltpu.make_async_remote_copy(
                        src_ref=red_ref.at[b],
                        dst_ref=ag_ref.at[b * N_DEV + i],
                        send_sem=ag_send.at[b * N_DEV + j],
                        recv_sem=ag_recv.at[b * N_DEV + i],
                        device_id=(j,),
                        device_id_type=pl.DeviceIdType.MESH,
                    )
                    rdma.start()

        for b in range(B):
            for j in range(N_DEV):
                @pl.when(i != j)
                def _():
                    rdma = pltpu.make_async_remote_copy(
                        src_ref=red_ref.at[b],
                        dst_ref=ag_ref.at[b * N_DEV + j],
                        send_sem=ag_send.at[b * N_DEV + j],
                        recv_sem=ag_recv.at[b * N_DEV + j],
                        device_id=(j,), device_id_type=pl.DeviceIdType.MESH,
                    )
                    rdma.wait_recv()
                    out_ref[b * SQ + j * HC:b * SQ + (j + 1) * HC, :] = (
                        ag_ref[b * N_DEV + j].astype(jnp.float32))

        for b in range(B):
            for j in range(N_DEV):
                @pl.when(i != j)
                def _():
                    r1 = pltpu.make_async_remote_copy(
                        src_ref=stage_ref.at[b * N_DEV + j],
                        dst_ref=rs_ref.at[b * N_DEV + j],
                        send_sem=rs_send.at[b * N_DEV + j],
                        recv_sem=rs_recv.at[b * N_DEV + j],
                        device_id=(j,), device_id_type=pl.DeviceIdType.MESH,
                    )
                    r1.wait_send()
                    r2 = pltpu.make_async_remote_copy(
                        src_ref=red_ref.at[b],
                        dst_ref=ag_ref.at[b * N_DEV + j],
                        send_sem=ag_send.at[b * N_DEV + j],
                        recv_sem=ag_recv.at[b * N_DEV + j],
                        device_id=(j,), device_id_type=pl.DeviceIdType.MESH,
                    )
                    r2.wait_send()

    out2 = pl.pallas_call(
        body,
        out_shape=jax.ShapeDtypeStruct((B * SQ, D), jnp.float32),
        in_specs=[pl.BlockSpec(memory_space=pltpu.VMEM)] * 5,
        out_specs=pl.BlockSpec(memory_space=pltpu.VMEM),
        scratch_shapes=[
            pltpu.VMEM((B * N_DEV, HC, D), jnp.bfloat16),
            pltpu.VMEM((B, HC, D), jnp.bfloat16),
            pltpu.VMEM((B * N_DEV, HC, D), jnp.bfloat16),
            pltpu.VMEM((B * N_DEV, HC, D), jnp.bfloat16),
            pltpu.SemaphoreType.DMA((B * N_DEV,)),
            pltpu.SemaphoreType.DMA((B * N_DEV,)),
            pltpu.SemaphoreType.DMA((B * N_DEV,)),
            pltpu.SemaphoreType.DMA((B * N_DEV,)),
        ],
        compiler_params=pltpu.CompilerParams(collective_id=0),
    )(x2, Wq, Wo, k_loc, v_loc)
    return out2.reshape(B, SQ, D)


# device time: 22899 ns/iter; 3.1347x vs baseline; 1.1043x over previous
import jax
import jax.numpy as jnp
from jax import lax
from jax.experimental import pallas as pl
from jax.experimental.pallas import tpu as pltpu

N_DEV = 8
B = 2
SQ = 256
D = 768
HQ_LOC = 8
HKV_LOC = 2
GROUP = 4
DH = 64
SKV = 512
HC = SQ // N_DEV
N_PEERS = N_DEV - 1
SCALE = 0.125


def kernel(x, Wq, Wo, K_ext, V_ext):
    my_i = lax.axis_index("i")
    k_flat = K_ext.reshape(B, SKV, 16 * DH)
    v_flat = V_ext.reshape(B, SKV, 16 * DH)
    start = my_i * (HKV_LOC * DH)
    k_loc = lax.dynamic_slice(k_flat, (0, 0, start), (B, SKV, HKV_LOC * DH))
    v_loc = lax.dynamic_slice(v_flat, (0, 0, start), (B, SKV, HKV_LOC * DH))
    x2 = x.reshape(B * SQ, D)

    def body(x_ref, wq_ref, wo_ref, k_ref, v_ref, out_ref,
             stage_ref, red_ref, rs_ref, ag_ref,
             rs_send, rs_recv, ag_send, ag_recv):
        i = lax.axis_index("i")

        barrier = pltpu.get_barrier_semaphore()
        for o in range(1, N_DEV):
            pl.semaphore_signal(barrier, inc=1,
                                device_id=(lax.rem(i + o, N_DEV),),
                                device_id_type=pl.DeviceIdType.MESH)

        wq = wq_ref[...].astype(jnp.bfloat16)
        wo = wo_ref[...].astype(jnp.bfloat16)

        def partial_batch(b):
            xb = x_ref[b * SQ:(b + 1) * SQ, :].astype(jnp.bfloat16)
            q = lax.dot_general(xb, wq, (((1,), (0,)), ((), ())),
                                preferred_element_type=jnp.float32)
            q = q.astype(jnp.bfloat16)
            cols = [None] * HQ_LOC
            for c in range(HKV_LOC):
                qg = jnp.concatenate(
                    [q[:, h * DH:(h + 1) * DH]
                     for h in range(c * GROUP, (c + 1) * GROUP)], axis=0)
                kc = k_ref[b, :, c * DH:(c + 1) * DH].astype(jnp.bfloat16)
                vc = v_ref[b, :, c * DH:(c + 1) * DH].astype(jnp.bfloat16)
                s = lax.dot_general(qg, kc, (((1,), (1,)), ((), ())),
                                    preferred_element_type=jnp.float32)
                s = s * SCALE
                m = jnp.max(s, axis=-1, keepdims=True)
                p = jnp.exp(s - m)
                l = jnp.sum(p, axis=-1, keepdims=True)
                o_ = lax.dot_general(p.astype(jnp.bfloat16), vc,
                                     (((1,), (0,)), ((), ())),
                                     preferred_element_type=jnp.float32)
                o_ = o_ / l
                for g in range(GROUP):
                    cols[c * GROUP + g] = o_[g * SQ:(g + 1) * SQ, :]
            attn_b = jnp.concatenate(cols, axis=1).astype(jnp.bfloat16)
            return lax.dot_general(attn_b, wo, (((1,), (0,)), ((), ())),
                                   preferred_element_type=jnp.float32)

        for b in range(B):
            part = partial_batch(b)
            for c in range(N_DEV):
                stage_ref[b * N_DEV + c] = part[c * HC:(c + 1) * HC, :].astype(
                    jnp.bfloat16)
            if b == 0:
                pl.semaphore_wait(barrier, N_PEERS)
            for c in range(N_DEV):
                @pl.when(i != c)
                def _():
                    rdma = pltpu.make_async_remote_copy(
                        src_ref=stage_ref.at[b * N_DEV + c],
                        dst_ref=rs_ref.at[b * N_DEV + i],
                        send_sem=rs_send.at[b * N_DEV + c],
                        recv_sem=rs_recv.at[b * N_DEV + i],
                        device_id=(c,),
                        device_id_type=pl.DeviceIdType.MESH,
                    )
                    rdma.start()

        for b in range(B):
            rs_ref[pl.ds(b * N_DEV + i, 1)] = stage_ref[pl.ds(b * N_DEV + i, 1)]
            for j in range(N_DEV):
                @pl.when(i != j)
                def _():
                    rdma = pltpu.make_async_remote_copy(
                        src_ref=stage_ref.at[b * N_DEV + j],
                        dst_ref=rs_ref.at[b * N_DEV + j],
                        send_sem=rs_send.at[b * N_DEV + j],
                        recv_sem=rs_recv.at[b * N_DEV + j],
                        device_id=(j,), device_id_type=pl.DeviceIdType.MESH,
                    )
                    rdma.wait_recv()
            red = rs_ref[b * N_DEV].astype(jnp.float32)
            for j in range(1, N_DEV):
                red = red + rs_ref[b * N_DEV + j].astype(jnp.float32)
            red_ref[b] = red.astype(jnp.bfloat16)
            out_ref[pl.ds(b * SQ + i * HC, HC), :] = red
            for j in range(N_DEV):
                @pl.when(i != j)
                def _():
                    rdma = pltpu.make_async_remote_copy(
                        src_ref=red_ref.at[b],
                        dst_ref=ag_ref.at[b * N_DEV + i],
                        send_sem=ag_send.at[b * N_DEV + j],
                        recv_sem=ag_recv.at[b * N_DEV + i],
                        device_id=(j,),
                        device_id_type=pl.DeviceIdType.MESH,
                    )
                    rdma.start()

        for b in range(B):
            for j in range(N_DEV):
                @pl.when(i != j)
                def _():
                    rdma = pltpu.make_async_remote_copy(
                        src_ref=red_ref.at[b],
                        dst_ref=ag_ref.at[b * N_DEV + j],
                        send_sem=ag_send.at[b * N_DEV + j],
                        recv_sem=ag_recv.at[b * N_DEV + j],
                        device_id=(j,), device_id_type=pl.DeviceIdType.MESH,
                    )
                    rdma.wait_recv()
                    out_ref[b * SQ + j * HC:b * SQ + (j + 1) * HC, :] = (
                        ag_ref[b * N_DEV + j].astype(jnp.float32))

        for b in range(B):
            for j in range(N_DEV):
                @pl.when(i != j)
                def _():
                    r1 = pltpu.make_async_remote_copy(
                        src_ref=stage_ref.at[b * N_DEV + j],
                        dst_ref=rs_ref.at[b * N_DEV + j],
                        send_sem=rs_send.at[b * N_DEV + j],
                        recv_sem=rs_recv.at[b * N_DEV + j],
                        device_id=(j,), device_id_type=pl.DeviceIdType.MESH,
                    )
                    r1.wait_send()
                    r2 = pltpu.make_async_remote_copy(
                        src_ref=red_ref.at[b],
                        dst_ref=ag_ref.at[b * N_DEV + j],
                        send_sem=ag_send.at[b * N_DEV + j],
                        recv_sem=ag_recv.at[b * N_DEV + j],
                        device_id=(j,), device_id_type=pl.DeviceIdType.MESH,
                    )
                    r2.wait_send()

    out2 = pl.pallas_call(
        body,
        out_shape=jax.ShapeDtypeStruct((B * SQ, D), jnp.float32),
        in_specs=[pl.BlockSpec(memory_space=pltpu.VMEM)] * 5,
        out_specs=pl.BlockSpec(memory_space=pltpu.VMEM),
        scratch_shapes=[
            pltpu.VMEM((B * N_DEV, HC, D), jnp.bfloat16),
            pltpu.VMEM((B, HC, D), jnp.bfloat16),
            pltpu.VMEM((B * N_DEV, HC, D), jnp.bfloat16),
            pltpu.VMEM((B * N_DEV, HC, D), jnp.bfloat16),
            pltpu.SemaphoreType.DMA((B * N_DEV,)),
            pltpu.SemaphoreType.DMA((B * N_DEV,)),
            pltpu.SemaphoreType.DMA((B * N_DEV,)),
            pltpu.SemaphoreType.DMA((B * N_DEV,)),
        ],
        compiler_params=pltpu.CompilerParams(collective_id=0),
    )(x2, Wq, Wo, k_loc, v_loc)
    return out2.reshape(B, SQ, D)
